# Initial kernel scaffold; baseline (speedup 1.0000x reference)
import jax
import jax.numpy as jnp
from jax.experimental import pallas as pl


def kernel(x, edge_index, W1, b1, W2, b2, W3, b3, W4, b4, W5, b5, W6, b6):
    raise NotImplementedError("write your pallas kernel here")


# speedup vs baseline: 2.3424x; 2.3424x over previous
import functools

import jax
import jax.numpy as jnp
from jax import lax
from jax.experimental import pallas as pl
from jax.experimental.pallas import tpu as pltpu
from jax.experimental.pallas import tpu_sc as plsc

N = 50000
E = 800000
DIN = 500
HID = 128

NP = 50176
KP = 512
EP = 802816
NC = 2
NS = 16

HROWS = 51200
HGARB = 50432
DEG_CHUNKS = 196

RNG = 12032
ACC_ROWS = 12160
AGG_ROWS = 72192
AGG_CHUNKS = 392
NPASS = 3

_F32 = jnp.float32


def _sc_mesh():
    return plsc.VectorSubcoreMesh(
        core_axis_name="c", subcore_axis_name="s", num_cores=NC, num_subcores=NS
    )


def _iota16():
    return lax.iota(jnp.int32, 16)


def _deg_body(edges_hbm, hist_hbm, srcb, dstb, idxb, ones, zb, hsp):
    c = lax.axis_index("c")
    s = lax.axis_index("s")
    t = c * NS + s

    def fill(j, carry):
        zb[j, :] = jnp.zeros((16,), _F32)
        ones[j, :] = jnp.ones((16,), _F32)
        return carry

    lax.fori_loop(0, 128, fill, 0)

    def chunk_idx(k):
        for v in range(8):
            idxb[pl.ds(v * 16, 16)] = (s * 25 + k) * 128 + v * 16 + _iota16()

    def zk(k, carry):
        chunk_idx(k)
        pltpu.sync_copy(zb, hsp.at[idxb])
        return carry

    lax.fori_loop(0, 25, zk, 0)
    plsc.subcore_barrier()

    def step(j, carry):
        off = t * 25088 + j * 128
        pltpu.sync_copy(edges_hbm.at[0, pl.ds(off, 128)], srcb)
        pltpu.sync_copy(edges_hbm.at[1, pl.ds(off, 128)], dstb)
        for v in range(8):
            sv = srcb[pl.ds(v * 16, 16)]
            dv = dstb[pl.ds(v * 16, 16)]
            g = HGARB + ((j * 128 + v * 16 + _iota16()) & 255)
            idxb[pl.ds(v * 16, 16)] = jnp.where(sv != dv, dv, g)
        pltpu.sync_copy(ones, hsp.at[idxb], add=True)
        return carry

    lax.fori_loop(0, DEG_CHUNKS, step, 0)
    plsc.subcore_barrier()

    def wk(k, carry):
        chunk_idx(k)
        pltpu.sync_copy(hsp.at[idxb], ones)
        pltpu.sync_copy(ones, hist_hbm.at[pl.ds(c * HROWS + (s * 25 + k) * 128, 128)])
        return carry

    lax.fori_loop(0, 25, wk, 0)


_deg_call = functools.partial(
    pl.kernel,
    out_type=jax.ShapeDtypeStruct((NC * HROWS, 16), _F32),
    mesh=_sc_mesh(),
    scratch_types=[
        pltpu.VMEM((128,), jnp.int32),
        pltpu.VMEM((128,), jnp.int32),
        pltpu.VMEM((128,), jnp.int32),
        pltpu.VMEM((128, 16), _F32),
        pltpu.VMEM((128, 16), _F32),
        pltpu.VMEM_SHARED((HROWS, 16), _F32),
    ],
)(_deg_body)


def _agg_body(hp_hbm, edges_hbm, agg_hbm, srcb, dstb, sidxb, rows, acc, sem):
    c = lax.axis_index("c")
    s = lax.axis_index("s")

    def do_pass(p, carry):
        def fillz(j, inner):
            for v in range(8):
                rows[j, pl.ds(v * 16, 16)] = jnp.zeros((16,), _F32)
            return inner

        lax.fori_loop(0, 128, fillz, 0)
        active = jnp.logical_or(p < NPASS - 1, c == 0)

        @pl.when(active)
        def _zero():
            def zk(k, inner):
                cid = s + 16 * k

                @pl.when(cid < 95)
                def _():
                    for v in range(8):
                        sidxb[pl.ds(v * 16, 16)] = cid * 128 + v * 16 + _iota16()
                    pltpu.sync_copy(rows, acc.at[sidxb])

                return inner

            lax.fori_loop(0, 6, zk, 0)

        plsc.subcore_barrier()
        rbase = (2 * p + c) * RNG

        def step(j, carry2):
            off = s * 50176 + j * 128
            pltpu.sync_copy(edges_hbm.at[0, pl.ds(off, 128)], srcb)
            pltpu.sync_copy(edges_hbm.at[1, pl.ds(off, 128)], dstb)
            cp = pltpu.async_copy(hp_hbm.at[srcb], rows, sem)
            for v in range(8):
                sv = srcb[pl.ds(v * 16, 16)]
                dv = dstb[pl.ds(v * 16, 16)]
                loc = dv - rbase
                ok = jnp.logical_and(sv != dv, jnp.logical_and(loc >= 0, loc < RNG))
                g = RNG + ((j * 128 + v * 16 + _iota16()) & 127)
                sidxb[pl.ds(v * 16, 16)] = jnp.where(ok, loc, g)
            cp.wait()
            pltpu.sync_copy(rows, acc.at[sidxb], add=True)
            return carry2

        @pl.when(active)
        def _scan():
            lax.fori_loop(0, AGG_CHUNKS, step, 0)

        plsc.subcore_barrier()

        @pl.when(active)
        def _writeout():
            def wk(k, inner):
                base = s * 752 + k * 128
                for v in range(8):
                    sidxb[pl.ds(v * 16, 16)] = base + v * 16 + _iota16()
                pltpu.sync_copy(acc.at[sidxb], rows)

                @pl.when(k < 5)
                def _():
                    pltpu.sync_copy(rows, agg_hbm.at[pl.ds(rbase + base, 128)])

                @pl.when(k == 5)
                def _():
                    pltpu.sync_copy(rows.at[pl.ds(0, 112)], agg_hbm.at[pl.ds(rbase + base, 112)])

                return inner

            lax.fori_loop(0, 6, wk, 0)

        plsc.subcore_barrier()
        return carry

    lax.fori_loop(0, NPASS, do_pass, 0)


_agg_call = functools.partial(
    pl.kernel,
    out_type=jax.ShapeDtypeStruct((AGG_ROWS, HID), _F32),
    mesh=_sc_mesh(),
    scratch_types=[
        pltpu.VMEM((128,), jnp.int32),
        pltpu.VMEM((128,), jnp.int32),
        pltpu.VMEM((128,), jnp.int32),
        pltpu.VMEM((128, HID), _F32),
        pltpu.VMEM_SHARED((ACC_ROWS, HID), _F32),
        pltpu.SemaphoreType.DMA,
    ],
)(_agg_body)


def _dinv_body(h0_ref, h1_ref, o_ref):
    deg = 1.0 + h0_ref[:, 0:1] + h1_ref[:, 0:1]
    o_ref[...] = lax.rsqrt(deg)


def _mm1_body(x_ref, w_ref, d_ref, o_ref):
    o_ref[...] = d_ref[...] * jnp.dot(x_ref[...], w_ref[...], preferred_element_type=_F32)


def _mid_body(a_ref, h_ref, d_ref, b_ref, w_ref, o_ref):
    u = d_ref[...] * (a_ref[...] + h_ref[...])
    e = jnp.maximum(u + b_ref[...], 0.0)
    o_ref[...] = d_ref[...] * jnp.dot(e, w_ref[...], preferred_element_type=_F32)


def _l5_body(a_ref, h_ref, d_ref, b_ref, o_ref):
    u = d_ref[...] * (a_ref[...] + h_ref[...])
    o_ref[...] = d_ref[...] * jnp.maximum(u + b_ref[...], 0.0)


def _fin_body(a_ref, h_ref, d_ref, w_ref, b_ref, o_ref):
    u = d_ref[...] * (a_ref[...] + h_ref[...])
    o_ref[...] = jnp.maximum(
        jnp.dot(u, w_ref[...], preferred_element_type=_F32) + b_ref[...], 0.0
    )


_GRID = (NP // 512,)


def _row_spec(w):
    return pl.BlockSpec((512, w), lambda i: (i, 0))


def _full_spec(h, w):
    return pl.BlockSpec((h, w), lambda i: (0, 0))


def _dinv_call(h0, h1):
    return pl.pallas_call(
        _dinv_body,
        grid=_GRID,
        in_specs=[_row_spec(16), _row_spec(16)],
        out_specs=_row_spec(1),
        out_shape=jax.ShapeDtypeStruct((NP, 1), _F32),
    )(h0, h1)


def _mm1_call(x_pad, w1, dinv):
    return pl.pallas_call(
        _mm1_body,
        grid=_GRID,
        in_specs=[_row_spec(KP), _full_spec(KP, HID), _row_spec(1)],
        out_specs=_row_spec(HID),
        out_shape=jax.ShapeDtypeStruct((NP, HID), _F32),
    )(x_pad, w1, dinv)


def _mid_call(agg, hp, dinv, b, w):
    return pl.pallas_call(
        _mid_body,
        grid=_GRID,
        in_specs=[_row_spec(HID), _row_spec(HID), _row_spec(1),
                  _full_spec(1, HID), _full_spec(HID, HID)],
        out_specs=_row_spec(HID),
        out_shape=jax.ShapeDtypeStruct((NP, HID), _F32),
    )(agg, hp, dinv, b, w)


def _l5_call(agg, hp, dinv, b):
    return pl.pallas_call(
        _l5_body,
        grid=_GRID,
        in_specs=[_row_spec(HID), _row_spec(HID), _row_spec(1), _full_spec(1, HID)],
        out_specs=_row_spec(HID),
        out_shape=jax.ShapeDtypeStruct((NP, HID), _F32),
    )(agg, hp, dinv, b)


def _fin_call(agg, hp, dinv, w6, b6):
    return pl.pallas_call(
        _fin_body,
        grid=_GRID,
        in_specs=[_row_spec(HID), _row_spec(HID), _row_spec(1),
                  _full_spec(HID, KP), _full_spec(1, KP)],
        out_specs=_row_spec(KP),
        out_shape=jax.ShapeDtypeStruct((NP, KP), _F32),
    )(agg, hp, dinv, w6, b6)


@jax.jit
def _impl(x, edge_index, W1, b1, W2, b2, W3, b3, W4, b4, W5, b5, W6, b6):
    edges = jnp.pad(edge_index, ((0, 0), (0, EP - E)))
    x_pad = jnp.pad(x, ((0, NP - N), (0, KP - DIN)))
    w1 = jnp.pad(W1, ((0, KP - DIN), (0, 0)))
    w6 = jnp.pad(W6, ((0, 0), (0, KP - DIN)))
    b6p = jnp.pad(b6, (0, KP - DIN)).reshape(1, KP)

    hist = _deg_call(edges)
    dinv = _dinv_call(hist[:HROWS], hist[HROWS:])

    hp = _mm1_call(x_pad, w1, dinv)
    for b, w in ((b1, W2), (b2, W3), (b3, W4), (b4, W5)):
        agg = _agg_call(hp, edges)
        hp = _mid_call(agg, hp, dinv, b.reshape(1, HID), w)
    agg = _agg_call(hp, edges)
    hp = _l5_call(agg, hp, dinv, b5.reshape(1, HID))
    agg = _agg_call(hp, edges)
    y = _fin_call(agg, hp, dinv, w6, b6p)
    return y[:N, :DIN]


def kernel(x, edge_index, W1, b1, W2, b2, W3, b3, W4, b4, W5, b5, W6, b6):
    return _impl(x, edge_index, W1, b1, W2, b2, W3, b3, W4, b4, W5, b5, W6, b6)


# speedup vs baseline: 3.1063x; 1.3261x over previous
import functools

import jax
import jax.numpy as jnp
from jax import lax
from jax.experimental import pallas as pl
from jax.experimental.pallas import tpu as pltpu
from jax.experimental.pallas import tpu_sc as plsc

N = 50000
E = 800000
DIN = 500
HID = 128

NP = 50176
KP = 512
EP = 802816
NC = 2
NS = 16

HROWS = 51200
HGARB = 50432
DEG_CHUNKS = 196

RNG = 7424
ACC_ROWS = 7552
AGG_ROWS = 59392
AGG_CHUNKS = 392
NPASS = 4

_F32 = jnp.float32


def _sc_mesh():
    return plsc.VectorSubcoreMesh(
        core_axis_name="c", subcore_axis_name="s", num_cores=NC, num_subcores=NS
    )


def _iota16():
    return lax.iota(jnp.int32, 16)


def _deg_body(edges_hbm, hist_hbm, srcb, dstb, idxb, ones, zb, hsp):
    c = lax.axis_index("c")
    s = lax.axis_index("s")
    t = c * NS + s

    def fill(j, carry):
        zb[j, :] = jnp.zeros((16,), _F32)
        ones[j, :] = jnp.ones((16,), _F32)
        return carry

    lax.fori_loop(0, 128, fill, 0)

    def chunk_idx(k):
        for v in range(8):
            idxb[pl.ds(v * 16, 16)] = (s * 25 + k) * 128 + v * 16 + _iota16()

    def zk(k, carry):
        chunk_idx(k)
        pltpu.sync_copy(zb, hsp.at[idxb])
        return carry

    lax.fori_loop(0, 25, zk, 0)
    plsc.subcore_barrier()

    def step(j, carry):
        off = t * 25088 + j * 128
        pltpu.sync_copy(edges_hbm.at[0, pl.ds(off, 128)], srcb)
        pltpu.sync_copy(edges_hbm.at[1, pl.ds(off, 128)], dstb)
        for v in range(8):
            sv = srcb[pl.ds(v * 16, 16)]
            dv = dstb[pl.ds(v * 16, 16)]
            g = HGARB + ((j * 128 + v * 16 + _iota16()) & 255)
            idxb[pl.ds(v * 16, 16)] = jnp.where(sv != dv, dv, g)
        pltpu.sync_copy(ones, hsp.at[idxb], add=True)
        return carry

    lax.fori_loop(0, DEG_CHUNKS, step, 0)
    plsc.subcore_barrier()

    def wk(k, carry):
        chunk_idx(k)
        pltpu.sync_copy(hsp.at[idxb], ones)
        pltpu.sync_copy(ones, hist_hbm.at[pl.ds(c * HROWS + (s * 25 + k) * 128, 128)])
        return carry

    lax.fori_loop(0, 25, wk, 0)


_deg_call = functools.partial(
    pl.kernel,
    out_type=jax.ShapeDtypeStruct((NC * HROWS, 16), _F32),
    mesh=_sc_mesh(),
    scratch_types=[
        pltpu.VMEM((128,), jnp.int32),
        pltpu.VMEM((128,), jnp.int32),
        pltpu.VMEM((128,), jnp.int32),
        pltpu.VMEM((128, 16), _F32),
        pltpu.VMEM((128, 16), _F32),
        pltpu.VMEM_SHARED((HROWS, 16), _F32),
    ],
)(_deg_body)


def _agg_body(hp_hbm, edges3_hbm, agg_hbm, ebuf, sidx4, rows4, acc,
              m0, m1, m2, m3):
    c = lax.axis_index("c")
    s = lax.axis_index("s")
    sem = (m0, m1, m2, m3)

    def gather_wait(b):
        pltpu.make_async_copy(hp_hbm, rows4.at[b], sem[b]).wait()

    def scatter_wait(b):
        pltpu.make_async_copy(rows4.at[b], acc.at[sidx4.at[b]], sem[b]).wait()

    def do_pass(p, carry):
        def fillz(j, inner):
            for v in range(8):
                rows4[0, j, pl.ds(v * 16, 16)] = jnp.zeros((16,), _F32)
            return inner

        lax.fori_loop(0, 128, fillz, 0)
        active = jnp.logical_or(p < NPASS - 1, c == 0)

        def _zero():
            def zk(k, inner):
                cid = s + 16 * k

                @pl.when(cid < 59)
                def _():
                    for v in range(8):
                        sidx4[0, pl.ds(v * 16, 16)] = cid * 128 + v * 16 + _iota16()
                    pltpu.sync_copy(rows4.at[0], acc.at[sidx4.at[0]])

                return inner

            lax.fori_loop(0, 4, zk, 0)

        pl.when(active)(_zero)
        plsc.subcore_barrier()
        rbase = (2 * p + c) * RNG
        ebase = s * 392

        def load_block(blk):
            par = jnp.bitwise_and(blk, 1)
            pltpu.sync_copy(edges3_hbm.at[0, pl.ds(ebase + blk * 8, 8)], ebuf.at[par, 0])
            pltpu.sync_copy(edges3_hbm.at[1, pl.ds(ebase + blk * 8, 8)], ebuf.at[par, 1])

        def gather_start(m, bm):
            mblk = m // 8
            mpar = jnp.bitwise_and(mblk, 1)
            mrow = lax.rem(m, 8)
            pltpu.make_async_copy(
                hp_hbm.at[ebuf.at[mpar, 0, mrow]], rows4.at[bm], sem[bm]
            ).start()

        def consume(n, b):
            gather_wait(b)
            par = jnp.bitwise_and(n // 8, 1)
            row = lax.rem(n, 8)
            for v in range(8):
                sv = ebuf[par, 0, row, pl.ds(v * 16, 16)]
                dv = ebuf[par, 1, row, pl.ds(v * 16, 16)]
                loc = dv - rbase
                ok = jnp.logical_and(sv != dv, jnp.logical_and(loc >= 0, loc < RNG))
                g = RNG + ((n * 128 + v * 16 + _iota16()) & 127)
                sidx4[b, pl.ds(v * 16, 16)] = jnp.where(ok, loc, g)
            pltpu.make_async_copy(
                rows4.at[b], acc.at[sidx4.at[b]], sem[b]
            ).start(add=True)

        def prefetch(n, b):
            m = n + 2
            bm = (b + 2) & 3

            @pl.when(m < AGG_CHUNKS)
            def _():
                @pl.when(n >= 2)
                def _():
                    scatter_wait(bm)

                @pl.when(lax.rem(m, 8) == 0)
                def _():
                    load_block(m // 8)

                gather_start(m, bm)

        def _pipeline():
            load_block(0)
            gather_start(0, 0)
            gather_start(1, 1)

            def body(o, inner):
                for b in range(4):
                    n = 4 * o + b
                    consume(n, b)
                    prefetch(n, b)
                return inner

            lax.fori_loop(0, AGG_CHUNKS // 4, body, 0)
            for b in range(4):
                scatter_wait(b)

        pl.when(active)(_pipeline)
        plsc.subcore_barrier()

        def _writeout():
            def wk(k, inner):
                base = s * 464 + k * 128
                for v in range(8):
                    sidx4[0, pl.ds(v * 16, 16)] = base + v * 16 + _iota16()
                pltpu.sync_copy(acc.at[sidx4.at[0]], rows4.at[0])

                @pl.when(k < 3)
                def _():
                    pltpu.sync_copy(rows4.at[0], agg_hbm.at[pl.ds(rbase + base, 128)])

                @pl.when(k == 3)
                def _():
                    pltpu.sync_copy(
                        rows4.at[0, pl.ds(0, 80)],
                        agg_hbm.at[pl.ds(rbase + base, 80)],
                    )

                return inner

            lax.fori_loop(0, 4, wk, 0)

        pl.when(active)(_writeout)
        plsc.subcore_barrier()
        return carry

    lax.fori_loop(0, NPASS, do_pass, 0)


_agg_call = functools.partial(
    pl.kernel,
    out_type=jax.ShapeDtypeStruct((AGG_ROWS, HID), _F32),
    mesh=_sc_mesh(),
    scratch_types=[
        pltpu.VMEM((2, 2, 8, 128), jnp.int32),
        pltpu.VMEM((4, 128), jnp.int32),
        pltpu.VMEM((4, 128, HID), _F32),
        pltpu.VMEM_SHARED((ACC_ROWS, HID), _F32),
    ] + [pltpu.SemaphoreType.DMA] * 4,
)(_agg_body)


def _dinv_body(h0_ref, h1_ref, o_ref):
    deg = 1.0 + h0_ref[:, 0:1] + h1_ref[:, 0:1]
    o_ref[...] = lax.rsqrt(deg)


def _mm1_body(x_ref, w_ref, d_ref, o_ref):
    o_ref[...] = d_ref[...] * jnp.dot(x_ref[...], w_ref[...], preferred_element_type=_F32)


def _mid_body(a_ref, h_ref, d_ref, b_ref, w_ref, o_ref):
    u = d_ref[...] * (a_ref[...] + h_ref[...])
    e = jnp.maximum(u + b_ref[...], 0.0)
    o_ref[...] = d_ref[...] * jnp.dot(e, w_ref[...], preferred_element_type=_F32)


def _l5_body(a_ref, h_ref, d_ref, b_ref, o_ref):
    u = d_ref[...] * (a_ref[...] + h_ref[...])
    o_ref[...] = d_ref[...] * jnp.maximum(u + b_ref[...], 0.0)


def _fin_body(a_ref, h_ref, d_ref, w_ref, b_ref, o_ref):
    u = d_ref[...] * (a_ref[...] + h_ref[...])
    o_ref[...] = jnp.maximum(
        jnp.dot(u, w_ref[...], preferred_element_type=_F32) + b_ref[...], 0.0
    )


_GRID = (NP // 512,)


def _row_spec(w):
    return pl.BlockSpec((512, w), lambda i: (i, 0))


def _full_spec(h, w):
    return pl.BlockSpec((h, w), lambda i: (0, 0))


def _dinv_call(h0, h1):
    return pl.pallas_call(
        _dinv_body,
        grid=_GRID,
        in_specs=[_row_spec(16), _row_spec(16)],
        out_specs=_row_spec(1),
        out_shape=jax.ShapeDtypeStruct((NP, 1), _F32),
    )(h0, h1)


def _mm1_call(x_pad, w1, dinv):
    return pl.pallas_call(
        _mm1_body,
        grid=_GRID,
        in_specs=[_row_spec(KP), _full_spec(KP, HID), _row_spec(1)],
        out_specs=_row_spec(HID),
        out_shape=jax.ShapeDtypeStruct((NP, HID), _F32),
    )(x_pad, w1, dinv)


def _mid_call(agg, hp, dinv, b, w):
    return pl.pallas_call(
        _mid_body,
        grid=_GRID,
        in_specs=[_row_spec(HID), _row_spec(HID), _row_spec(1),
                  _full_spec(1, HID), _full_spec(HID, HID)],
        out_specs=_row_spec(HID),
        out_shape=jax.ShapeDtypeStruct((NP, HID), _F32),
    )(agg, hp, dinv, b, w)


def _l5_call(agg, hp, dinv, b):
    return pl.pallas_call(
        _l5_body,
        grid=_GRID,
        in_specs=[_row_spec(HID), _row_spec(HID), _row_spec(1), _full_spec(1, HID)],
        out_specs=_row_spec(HID),
        out_shape=jax.ShapeDtypeStruct((NP, HID), _F32),
    )(agg, hp, dinv, b)


def _fin_call(agg, hp, dinv, w6, b6):
    return pl.pallas_call(
        _fin_body,
        grid=_GRID,
        in_specs=[_row_spec(HID), _row_spec(HID), _row_spec(1),
                  _full_spec(HID, KP), _full_spec(1, KP)],
        out_specs=_row_spec(KP),
        out_shape=jax.ShapeDtypeStruct((NP, KP), _F32),
    )(agg, hp, dinv, w6, b6)


@jax.jit
def _impl(x, edge_index, W1, b1, W2, b2, W3, b3, W4, b4, W5, b5, W6, b6):
    edges = jnp.pad(edge_index, ((0, 0), (0, EP - E)))
    edges3 = edges.reshape(2, EP // 128, 128)
    x_pad = jnp.pad(x, ((0, NP - N), (0, KP - DIN)))
    w1 = jnp.pad(W1, ((0, KP - DIN), (0, 0)))
    w6 = jnp.pad(W6, ((0, 0), (0, KP - DIN)))
    b6p = jnp.pad(b6, (0, KP - DIN)).reshape(1, KP)

    hist = _deg_call(edges)
    dinv = _dinv_call(hist[:HROWS], hist[HROWS:])

    hp = _mm1_call(x_pad, w1, dinv)
    for b, w in ((b1, W2), (b2, W3), (b3, W4), (b4, W5)):
        agg = _agg_call(hp, edges3)
        hp = _mid_call(agg, hp, dinv, b.reshape(1, HID), w)
    agg = _agg_call(hp, edges3)
    hp = _l5_call(agg, hp, dinv, b5.reshape(1, HID))
    agg = _agg_call(hp, edges3)
    y = _fin_call(agg, hp, dinv, w6, b6p)
    return y[:N, :DIN]


def kernel(x, edge_index, W1, b1, W2, b2, W3, b3, W4, b4, W5, b5, W6, b6):
    return _impl(x, edge_index, W1, b1, W2, b2, W3, b3, W4, b4, W5, b5, W6, b6)
